# baseline (speedup 1.0000x reference)
import jax
import jax.numpy as jnp
from jax import lax
from jax.experimental import pallas as pl
from jax.experimental.pallas import tpu as pltpu

RATE = 1.0
B, C_IN, H, W = 8, 192, 224, 224
C_OUT = 192
K = int(C_OUT * RATE)
SPATIAL = H * W
CB = 32
NCB = C_IN // CB
NH = 2
HB = H // NH


def _fused_kernel(x_ref, w_ref, b_ref, t_ref, sp_acc):
    b = pl.program_id(0)
    c = pl.program_id(1)
    h = pl.program_id(2)
    a = jnp.abs(x_ref[...])
    part = jnp.sum(a, axis=2)

    @pl.when(h == 0)
    def _first():
        sp_acc[pl.ds(b, 1), pl.ds(c * CB, CB), :] = part

    @pl.when(h > 0)
    def _rest():
        sp_acc[pl.ds(b, 1), pl.ds(c * CB, CB), :] += part

    @pl.when((b == B - 1) & (c == NCB - 1) & (h == NH - 1))
    def _finish():
        s = jnp.sum(sp_acc[...], axis=2) * (1.0 / SPATIAL)
        s_ext = jnp.concatenate(
            [s, jnp.full((B, 1), RATE, jnp.float32)], axis=1)
        g = lax.dot_general(s_ext, w_ref[...], (((1,), (1,)), ((), ())),
                            preferred_element_type=jnp.float32)
        g = jnp.maximum(g + b_ref[...], 0.0)
        ge = g[:, :, None]
        gm = g[:, None, :]
        e_idx = lax.broadcasted_iota(jnp.int32, (B, C_OUT, C_OUT), 1)
        m_idx = lax.broadcasted_iota(jnp.int32, (B, C_OUT, C_OUT), 2)
        smaller = (gm < ge) | ((gm == ge) & (m_idx < e_idx))
        rank = jnp.sum(smaller.astype(jnp.int32), axis=2)
        t = jnp.where(rank >= K, g, 0.0)
        t_sum = jnp.sum(t, axis=1, keepdims=True)
        t_ref[...] = t / t_sum * C_OUT


@jax.jit
def kernel(x, gate_w, gate_b):
    t = pl.pallas_call(
        _fused_kernel,
        grid=(B, NCB, NH),
        in_specs=[
            pl.BlockSpec((1, CB, HB, W), lambda b, c, h: (b, c, h, 0)),
            pl.BlockSpec((C_OUT, C_IN + 1), lambda b, c, h: (0, 0)),
            pl.BlockSpec((1, C_OUT), lambda b, c, h: (0, 0)),
        ],
        out_specs=pl.BlockSpec((B, C_OUT), lambda b, c, h: (0, 0)),
        out_shape=jax.ShapeDtypeStruct((B, C_OUT), jnp.float32),
        scratch_shapes=[pltpu.VMEM((B, C_IN, W), jnp.float32)],
    )(x, gate_w, gate_b.reshape(1, C_OUT))
    return t[:, :, None, None]


# speedup vs baseline: 1.2714x; 1.2714x over previous
import jax
import jax.numpy as jnp
from jax import lax
from jax.experimental import pallas as pl
from jax.experimental.pallas import tpu as pltpu

RATE = 1.0
B, C_IN, H, W = 8, 192, 224, 224
C_OUT = 192
K = int(C_OUT * RATE)
SPATIAL = H * W
CB = 32
NCB = C_IN // CB


def _fused_kernel(x_ref, w_ref, b_ref, t_ref, sp_acc):
    b = pl.program_id(0)
    c = pl.program_id(1)
    a = jnp.abs(x_ref[...])
    sp_acc[pl.ds(b, 1), pl.ds(c * CB, CB), :] = jnp.sum(a, axis=2)

    @pl.when((b == B - 1) & (c == NCB - 1))
    def _finish():
        s = jnp.sum(sp_acc[...], axis=2) * (1.0 / SPATIAL)
        s_ext = jnp.concatenate(
            [s, jnp.full((B, 1), RATE, jnp.float32)], axis=1)
        g = lax.dot_general(s_ext, w_ref[...], (((1,), (1,)), ((), ())),
                            preferred_element_type=jnp.float32)
        g = jnp.maximum(g + b_ref[...], 0.0)
        ge = g[:, :, None]
        gm = g[:, None, :]
        e_idx = lax.broadcasted_iota(jnp.int32, (B, C_OUT, C_OUT), 1)
        m_idx = lax.broadcasted_iota(jnp.int32, (B, C_OUT, C_OUT), 2)
        smaller = (gm < ge) | ((gm == ge) & (m_idx < e_idx))
        rank = jnp.sum(smaller.astype(jnp.int32), axis=2)
        t = jnp.where(rank >= K, g, 0.0)
        t_sum = jnp.sum(t, axis=1, keepdims=True)
        t_ref[...] = t / t_sum * C_OUT


@jax.jit
def kernel(x, gate_w, gate_b):
    t = pl.pallas_call(
        _fused_kernel,
        grid=(B, NCB),
        in_specs=[
            pl.BlockSpec((1, CB, H, W), lambda b, c: (b, c, 0, 0)),
            pl.BlockSpec((C_OUT, C_IN + 1), lambda b, c: (0, 0)),
            pl.BlockSpec((1, C_OUT), lambda b, c: (0, 0)),
        ],
        out_specs=pl.BlockSpec((B, C_OUT), lambda b, c: (0, 0)),
        out_shape=jax.ShapeDtypeStruct((B, C_OUT), jnp.float32),
        scratch_shapes=[pltpu.VMEM((B, C_IN, W), jnp.float32)],
    )(x, gate_w, gate_b.reshape(1, C_OUT))
    return t[:, :, None, None]
